# baseline (speedup 1.0000x reference)
import functools

import jax
import jax.numpy as jnp
from jax import lax
from jax.experimental import pallas as pl
from jax.experimental.pallas import tpu as pltpu
from jax.experimental.pallas import tpu_sc as plsc

N = 10000
D = 128
R = 4
E = 320000
BN_EPS = 1e-5

NC = 2
NS = 16
L = 16

CH_A = 2528
CH_B = 2496
NTOT = CH_A + 3 * CH_B
B = 64
NSLOT = 3
SB = 512
NSB = 40
EP = NSB * SB
EPAD = NS * EP
GBASE = CH_A * R
SH_ROWS = GBASE + 64
ZROWS = SH_ROWS // NS
SHIFT = 14


def _lane_take(v, idx):
    return lax.gather(
        v, idx[:, None],
        lax.GatherDimensionNumbers(
            offset_dims=(), collapsed_slice_dims=(0,), start_index_map=(0,)),
        slice_sizes=(1,),
        mode=lax.GatherScatterMode.PROMISE_IN_BOUNDS)


def _sc_agg_kernel(x_hbm, src_hbm, gid_hbm, z_hbm, out_hbm,
                   srcm0, srcm1, gidm0, gidm1, packedc, rows0, rows1, rows2,
                   gb0, gb1, gb2, sb0, sb1, sb2, shared,
                   semm0, semm1, semg0, semg1, semg2, sems0, sems1, sems2):
    c = lax.axis_index("c")
    s = lax.axis_index("s")
    srcm = (srcm0, srcm1)
    gidm = (gidm0, gidm1)
    rows = (rows0, rows1, rows2)
    gbuf = (gb0, gb1, gb2)
    sbuf = (sb0, sb1, sb2)
    semm = (semm0, semm1)
    semg = (semg0, semg1, semg2)
    sems = (sems0, sems1, sems2)
    iota = lax.iota(jnp.int32, L)

    def meta_start(sb, par):
        pltpu.async_copy(src_hbm.at[s, pl.ds(sb * SB, SB)], srcm[par],
                         semm[par])
        pltpu.async_copy(gid_hbm.at[s, pl.ds(sb * SB, SB)], gidm[par],
                         semm[par])

    def meta_wait(par):
        pltpu.make_async_copy(src_hbm.at[0, pl.ds(0, SB)], srcm[par],
                              semm[par]).wait()
        pltpu.make_async_copy(gid_hbm.at[0, pl.ds(0, SB)], gidm[par],
                              semm[par]).wait()

    def gather_start(jpar):
        pltpu.async_copy(x_hbm.at[gbuf[jpar].at[0]], rows[jpar], semg[jpar])

    def gather_wait(jpar):
        pltpu.make_async_copy(x_hbm.at[gbuf[jpar].at[0]], rows[jpar],
                              semg[jpar]).wait()

    def prep(b, jpar):
        for jj in range(B // L):
            pv = packedc[pl.ds(b * B + jj * L, L)]
            gbuf[jpar][0, pl.ds(jj * L, L)] = lax.shift_right_logical(
                pv, SHIFT)
            sbuf[jpar][0, pl.ds(jj * L, L)] = pv & ((1 << SHIFT) - 1)

    def scatter_start(jpar):
        pltpu.async_copy(rows[jpar], shared.at[sbuf[jpar].at[0]], sems[jpar],
                         add=True)

    def scatter_wait(jpar):
        pltpu.make_async_copy(rows[jpar], shared.at[sbuf[jpar].at[0]],
                              sems[jpar]).wait()

    def run_pass(lo4, acc_rows, out_base):
        orows = acc_rows // NS
        pltpu.sync_copy(z_hbm, shared.at[pl.ds(s * ZROWS, ZROWS)])
        plsc.subcore_barrier()

        meta_start(0, 0)

        def compact_outer(q, fill):
            for par in (0, 1):
                sb = 2 * q + par
                meta_wait(par)
                if par == 0:
                    meta_start(sb + 1, 1)
                else:
                    @pl.when(q < NSB // 2 - 1)
                    def _():
                        meta_start(sb + 1, 0)
                def one_group(g):
                    gv = gidm[par][pl.ds(g * L, L)]
                    sv = srcm[par][pl.ds(g * L, L)]
                    rel = gv - lo4
                    inr = (rel >= 0) & (rel < acc_rows)
                    packed = lax.shift_left(sv, SHIFT) | (
                        rel & ((1 << SHIFT) - 1))
                    ps = jnp.where(inr, jnp.full((L,), 1, jnp.int32),
                                   jnp.full((L,), 0, jnp.int32))
                    for kk in (1, 2, 4, 8):
                        sh = _lane_take(ps, jnp.maximum(iota - kk, 0))
                        ps = ps + jnp.where(iota >= kk, sh, 0)
                    tgt = iota + 1
                    lo = jnp.zeros((L,), jnp.int32)
                    for kk in (8, 4, 2, 1):
                        sm = _lane_take(ps, lo + (kk - 1))
                        lo = jnp.where(sm < tgt, lo + kk, lo)
                    perm = jnp.minimum(lo, L - 1)
                    return _lane_take(packed, perm), ps

                def group_body(q2, fill):
                    g = 2 * q2
                    comp0, ps0 = one_group(g)
                    comp1, ps1 = one_group(g + 1)
                    packedc[pl.ds(fill, L)] = comp0
                    fill = fill + ps0[15]
                    packedc[pl.ds(fill, L)] = comp1
                    return fill + ps1[15]

                fill = lax.fori_loop(0, SB // L // 2, group_body, fill)
            return fill

        fill = lax.fori_loop(0, NSB // 2, compact_outer, jnp.int32(0))

        for jj in range(B // L):
            packedc[pl.ds(fill + jj * L, L)] = (
                GBASE + jj * L + iota)

        nb = lax.div(fill + (B - 1), B)

        def ring(g, carry):
            b0 = NSLOT * g
            for t in range(NSLOT):
                b = b0 + t

                @pl.when(b < nb)
                def _():
                    @pl.when(b >= NSLOT)
                    def _():
                        scatter_wait(t)

                    prep(b, t)
                    gather_start(t)

            for t in range(NSLOT):
                b = b0 + t

                @pl.when(b < nb)
                def _():
                    gather_wait(t)
                    scatter_start(t)

            return carry

        lax.fori_loop(0, lax.div(nb + (NSLOT - 1), NSLOT), ring, 0)
        for t in range(NSLOT):
            @pl.when(nb > t)
            def _():
                scatter_wait(t)
        plsc.subcore_barrier()

        pltpu.sync_copy(
            shared.at[pl.ds(s * orows, orows)],
            out_hbm.at[pl.ds(out_base + s * orows, orows)])
        plsc.subcore_barrier()

    @pl.when(c == 0)
    def _():
        run_pass(0, CH_A * R, 0)

    @pl.when(c == 1)
    def _():
        run_pass(CH_A * R, CH_B * R, CH_A * R)
        run_pass((CH_A + CH_B) * R, CH_B * R, (CH_A + CH_B) * R)
        run_pass((CH_A + 2 * CH_B) * R, CH_B * R, (CH_A + 2 * CH_B) * R)


@functools.lru_cache(maxsize=None)
def _make_sc_agg():
    return functools.partial(
        pl.kernel,
        out_type=jax.ShapeDtypeStruct((NTOT * R, D), jnp.float32),
        mesh=plsc.VectorSubcoreMesh(core_axis_name="c", subcore_axis_name="s"),
        scratch_types=[
            pltpu.VMEM((SB,), jnp.int32),
            pltpu.VMEM((SB,), jnp.int32),
            pltpu.VMEM((SB,), jnp.int32),
            pltpu.VMEM((SB,), jnp.int32),
            pltpu.VMEM((EP + B,), jnp.int32),
            pltpu.VMEM((B, D), jnp.float32),
            pltpu.VMEM((B, D), jnp.float32),
            pltpu.VMEM((B, D), jnp.float32),
            pltpu.VMEM((1, B), jnp.int32),
            pltpu.VMEM((1, B), jnp.int32),
            pltpu.VMEM((1, B), jnp.int32),
            pltpu.VMEM((1, B), jnp.int32),
            pltpu.VMEM((1, B), jnp.int32),
            pltpu.VMEM((1, B), jnp.int32),
            pltpu.VMEM_SHARED((SH_ROWS, D), jnp.float32),
            pltpu.SemaphoreType.DMA,
            pltpu.SemaphoreType.DMA,
            pltpu.SemaphoreType.DMA,
            pltpu.SemaphoreType.DMA,
            pltpu.SemaphoreType.DMA,
            pltpu.SemaphoreType.DMA,
            pltpu.SemaphoreType.DMA,
            pltpu.SemaphoreType.DMA,
        ],
    )(_sc_agg_kernel)


BLK = 400
NB = N // BLK


def _tc_stats_kernel(x_ref, agg_ref, w1_ref, b1_ref, h_ref, st_ref):
    nb = pl.program_id(0)
    xb = x_ref[...]
    parts = []
    for r in range(R):
        xa = xb + agg_ref[:, r, :]
        h = jnp.dot(xa, w1_ref[r], preferred_element_type=jnp.float32) \
            + b1_ref[r]
        h_ref[:, r, :] = h
        ssum = jnp.sum(h, axis=0, keepdims=True)
        ssq = jnp.sum(h * h, axis=0, keepdims=True)
        parts.append(jnp.concatenate(
            [ssum, ssq, jnp.zeros((6, D), jnp.float32)], axis=0))
    contrib = jnp.stack(parts, axis=0)

    @pl.when(nb == 0)
    def _():
        st_ref[...] = contrib

    @pl.when(nb > 0)
    def _():
        st_ref[...] = st_ref[...] + contrib


def _tc_final_kernel(x_ref, h_ref, st_ref, ws_ref, w2_ref, pp_ref, o_ref):
    acc = jnp.dot(x_ref[...], ws_ref[...],
                  preferred_element_type=jnp.float32) + pp_ref[0, 3:4, :]
    inv_n = jnp.float32(1.0 / N)
    for r in range(R):
        mu = st_ref[r, 0:1, :] * inv_n
        var = st_ref[r, 1:2, :] * inv_n - mu * mu
        scale = lax.rsqrt(var + BN_EPS) * pp_ref[r, 0:1, :]
        hn = (h_ref[:, r, :] - mu) * scale + pp_ref[r, 1:2, :]
        hn = jnp.maximum(hn, 0.0)
        acc = acc + jnp.dot(hn, w2_ref[r],
                            preferred_element_type=jnp.float32)
        acc = acc + pp_ref[r, 2:3, :]
    o_ref[...] = acc


def kernel(x, edge_index, edge_type, W_self, b_self, W1, b1, gamma, beta,
           W2, b2):
    x = x.astype(jnp.float32)
    src = edge_index[0].astype(jnp.int32)
    dst = edge_index[1].astype(jnp.int32)
    typ = edge_type.astype(jnp.int32)

    pad = EPAD - E
    gid = dst * R + typ
    src = jnp.concatenate([src, jnp.zeros((pad,), jnp.int32)])
    gid = jnp.concatenate([gid, jnp.full((pad,), N * R, jnp.int32)])
    srcT = src.reshape(NS, EP)
    gidT = gid.reshape(NS, EP)

    zeros_hbm = jnp.zeros((ZROWS, D), jnp.float32)

    agg = _make_sc_agg()(x, srcT, gidT, zeros_hbm)
    agg3 = agg.reshape(NTOT, R, D)

    b1r = b1.reshape(R, 1, D)
    h, stats = pl.pallas_call(
        _tc_stats_kernel,
        grid=(NB,),
        in_specs=[
            pl.BlockSpec((BLK, D), lambda nb: (nb, 0)),
            pl.BlockSpec((BLK, R, D), lambda nb: (nb, 0, 0)),
            pl.BlockSpec((R, D, D), lambda nb: (0, 0, 0)),
            pl.BlockSpec((R, 1, D), lambda nb: (0, 0, 0)),
        ],
        out_specs=[
            pl.BlockSpec((BLK, R, D), lambda nb: (nb, 0, 0)),
            pl.BlockSpec((R, 8, D), lambda nb: (0, 0, 0)),
        ],
        out_shape=[
            jax.ShapeDtypeStruct((N, R, D), jnp.float32),
            jax.ShapeDtypeStruct((R, 8, D), jnp.float32),
        ],
    )(x, agg3, W1, b1r)

    params = jnp.zeros((R, 8, D), jnp.float32)
    params = params.at[:, 0, :].set(gamma)
    params = params.at[:, 1, :].set(beta)
    params = params.at[:, 2, :].set(b2)
    params = params.at[0, 3, :].set(b_self)

    out = pl.pallas_call(
        _tc_final_kernel,
        grid=(NB,),
        in_specs=[
            pl.BlockSpec((BLK, D), lambda nb: (nb, 0)),
            pl.BlockSpec((BLK, R, D), lambda nb: (nb, 0, 0)),
            pl.BlockSpec((R, 8, D), lambda nb: (0, 0, 0)),
            pl.BlockSpec((D, D), lambda nb: (0, 0)),
            pl.BlockSpec((R, D, D), lambda nb: (0, 0, 0)),
            pl.BlockSpec((R, 8, D), lambda nb: (0, 0, 0)),
        ],
        out_specs=pl.BlockSpec((BLK, D), lambda nb: (nb, 0)),
        out_shape=jax.ShapeDtypeStruct((N, D), jnp.float32),
    )(x, h, stats, W_self, W2, params)

    return out


# speedup vs baseline: 1.0391x; 1.0391x over previous
import functools

import jax
import jax.numpy as jnp
from jax import lax
from jax.experimental import pallas as pl
from jax.experimental.pallas import tpu as pltpu
from jax.experimental.pallas import tpu_sc as plsc

N = 10000
D = 128
R = 4
E = 320000
BN_EPS = 1e-5

NC = 2
NS = 16
L = 16

CH_A = 2528
CH_B = 2496
NTOT = CH_A + 3 * CH_B
B = 64
NSLOT = 3
SB = 512
NSB = 40
EP = NSB * SB
EPAD = NS * EP
GBASE = CH_A * R
SH_ROWS = GBASE + 64
ZROWS = SH_ROWS // NS
SHIFT = 14


def _lane_take(v, idx):
    return lax.gather(
        v, idx[:, None],
        lax.GatherDimensionNumbers(
            offset_dims=(), collapsed_slice_dims=(0,), start_index_map=(0,)),
        slice_sizes=(1,),
        mode=lax.GatherScatterMode.PROMISE_IN_BOUNDS)


def _sc_agg_kernel(x_hbm, src_hbm, gid_hbm, z_hbm, out_hbm,
                   srcm0, srcm1, gidm0, gidm1, packedc, rows0, rows1, rows2,
                   gb0, gb1, gb2, sb0, sb1, sb2, shared,
                   semm0, semm1, semg0, semg1, semg2, sems0, sems1, sems2):
    c = lax.axis_index("c")
    s = lax.axis_index("s")
    srcm = (srcm0, srcm1)
    gidm = (gidm0, gidm1)
    rows = (rows0, rows1, rows2)
    gbuf = (gb0, gb1, gb2)
    sbuf = (sb0, sb1, sb2)
    semm = (semm0, semm1)
    semg = (semg0, semg1, semg2)
    sems = (sems0, sems1, sems2)
    iota = lax.iota(jnp.int32, L)

    def meta_start(sb, par):
        pltpu.async_copy(src_hbm.at[s, pl.ds(sb * SB, SB)], srcm[par],
                         semm[par])
        pltpu.async_copy(gid_hbm.at[s, pl.ds(sb * SB, SB)], gidm[par],
                         semm[par])

    def meta_wait(par):
        pltpu.make_async_copy(src_hbm.at[0, pl.ds(0, SB)], srcm[par],
                              semm[par]).wait()
        pltpu.make_async_copy(gid_hbm.at[0, pl.ds(0, SB)], gidm[par],
                              semm[par]).wait()

    def gather_start(jpar):
        pltpu.async_copy(x_hbm.at[gbuf[jpar].at[0]], rows[jpar], semg[jpar])

    def gather_wait(jpar):
        pltpu.make_async_copy(x_hbm.at[gbuf[jpar].at[0]], rows[jpar],
                              semg[jpar]).wait()

    def prep(b, jpar):
        for jj in range(B // L):
            pv = packedc[pl.ds(b * B + jj * L, L)]
            gbuf[jpar][0, pl.ds(jj * L, L)] = lax.shift_right_logical(
                pv, SHIFT)
            sbuf[jpar][0, pl.ds(jj * L, L)] = pv & ((1 << SHIFT) - 1)

    def scatter_start(jpar):
        pltpu.async_copy(rows[jpar], shared.at[sbuf[jpar].at[0]], sems[jpar],
                         add=True)

    def scatter_wait(jpar):
        pltpu.make_async_copy(rows[jpar], shared.at[sbuf[jpar].at[0]],
                              sems[jpar]).wait()

    def run_pass(lo4, acc_rows, out_base):
        orows = acc_rows // NS
        pltpu.sync_copy(z_hbm, shared.at[pl.ds(s * ZROWS, ZROWS)])
        plsc.subcore_barrier()

        meta_start(0, 0)

        def compact_outer(q, fill):
            for par in (0, 1):
                sb = 2 * q + par
                meta_wait(par)
                if par == 0:
                    meta_start(sb + 1, 1)
                else:
                    @pl.when(q < NSB // 2 - 1)
                    def _():
                        meta_start(sb + 1, 0)
                def one_group(g):
                    gv = gidm[par][pl.ds(g * L, L)]
                    sv = srcm[par][pl.ds(g * L, L)]
                    rel = gv - lo4
                    inr = (rel >= 0) & (rel < acc_rows)
                    packed = lax.shift_left(sv, SHIFT) | (
                        rel & ((1 << SHIFT) - 1))
                    ps = jnp.where(inr, jnp.full((L,), 1, jnp.int32),
                                   jnp.full((L,), 0, jnp.int32))
                    for kk in (1, 2, 4, 8):
                        sh = _lane_take(ps, jnp.maximum(iota - kk, 0))
                        ps = ps + jnp.where(iota >= kk, sh, 0)
                    tgt = iota + 1
                    lo = jnp.zeros((L,), jnp.int32)
                    for kk in (8, 4, 2, 1):
                        sm = _lane_take(ps, lo + (kk - 1))
                        lo = jnp.where(sm < tgt, lo + kk, lo)
                    perm = jnp.minimum(lo, L - 1)
                    return _lane_take(packed, perm), ps

                def group_body(q2, fill):
                    g = 2 * q2
                    comp0, ps0 = one_group(g)
                    comp1, ps1 = one_group(g + 1)
                    packedc[pl.ds(fill, L)] = comp0
                    fill = fill + ps0[15]
                    packedc[pl.ds(fill, L)] = comp1
                    return fill + ps1[15]

                fill = lax.fori_loop(0, SB // L // 2, group_body, fill)
            return fill

        fill = lax.fori_loop(0, NSB // 2, compact_outer, jnp.int32(0))

        for jj in range(B // L):
            packedc[pl.ds(fill + jj * L, L)] = (
                GBASE + jj * L + iota)

        nb = lax.div(fill + (B - 1), B)

        def ring(g, carry):
            b0 = NSLOT * g
            for t in range(NSLOT):
                b = b0 + t

                @pl.when(b < nb)
                def _():
                    @pl.when(b >= NSLOT)
                    def _():
                        scatter_wait(t)

                    prep(b, t)
                    gather_start(t)

            for t in range(NSLOT):
                b = b0 + t

                @pl.when(b < nb)
                def _():
                    gather_wait(t)
                    scatter_start(t)

            return carry

        lax.fori_loop(0, lax.div(nb + (NSLOT - 1), NSLOT), ring, 0)
        for t in range(NSLOT):
            @pl.when(nb > t)
            def _():
                scatter_wait(t)
        plsc.subcore_barrier()

        pltpu.sync_copy(
            shared.at[pl.ds(s * orows, orows)],
            out_hbm.at[pl.ds(out_base + s * orows, orows)])
        plsc.subcore_barrier()

    @pl.when(c == 1)
    def _():
        run_pass(0, CH_A * R, 0)

    @pl.when(c == 0)
    def _():
        run_pass(CH_A * R, CH_B * R, CH_A * R)
        run_pass((CH_A + CH_B) * R, CH_B * R, (CH_A + CH_B) * R)
        run_pass((CH_A + 2 * CH_B) * R, CH_B * R, (CH_A + 2 * CH_B) * R)


@functools.lru_cache(maxsize=None)
def _make_sc_agg():
    return functools.partial(
        pl.kernel,
        out_type=jax.ShapeDtypeStruct((NTOT * R, D), jnp.float32),
        mesh=plsc.VectorSubcoreMesh(core_axis_name="c", subcore_axis_name="s"),
        scratch_types=[
            pltpu.VMEM((SB,), jnp.int32),
            pltpu.VMEM((SB,), jnp.int32),
            pltpu.VMEM((SB,), jnp.int32),
            pltpu.VMEM((SB,), jnp.int32),
            pltpu.VMEM((EP + B,), jnp.int32),
            pltpu.VMEM((B, D), jnp.float32),
            pltpu.VMEM((B, D), jnp.float32),
            pltpu.VMEM((B, D), jnp.float32),
            pltpu.VMEM((1, B), jnp.int32),
            pltpu.VMEM((1, B), jnp.int32),
            pltpu.VMEM((1, B), jnp.int32),
            pltpu.VMEM((1, B), jnp.int32),
            pltpu.VMEM((1, B), jnp.int32),
            pltpu.VMEM((1, B), jnp.int32),
            pltpu.VMEM_SHARED((SH_ROWS, D), jnp.float32),
            pltpu.SemaphoreType.DMA,
            pltpu.SemaphoreType.DMA,
            pltpu.SemaphoreType.DMA,
            pltpu.SemaphoreType.DMA,
            pltpu.SemaphoreType.DMA,
            pltpu.SemaphoreType.DMA,
            pltpu.SemaphoreType.DMA,
            pltpu.SemaphoreType.DMA,
        ],
    )(_sc_agg_kernel)


BLK = 400
NB = N // BLK


def _tc_stats_kernel(x_ref, agg_ref, w1_ref, b1_ref, h_ref, st_ref):
    nb = pl.program_id(0)
    xb = x_ref[...]
    parts = []
    for r in range(R):
        xa = xb + agg_ref[:, r, :]
        h = jnp.dot(xa, w1_ref[r], preferred_element_type=jnp.float32) \
            + b1_ref[r]
        h_ref[:, r, :] = h
        ssum = jnp.sum(h, axis=0, keepdims=True)
        ssq = jnp.sum(h * h, axis=0, keepdims=True)
        parts.append(jnp.concatenate(
            [ssum, ssq, jnp.zeros((6, D), jnp.float32)], axis=0))
    contrib = jnp.stack(parts, axis=0)

    @pl.when(nb == 0)
    def _():
        st_ref[...] = contrib

    @pl.when(nb > 0)
    def _():
        st_ref[...] = st_ref[...] + contrib


def _tc_final_kernel(x_ref, h_ref, st_ref, ws_ref, w2_ref, pp_ref, o_ref):
    acc = jnp.dot(x_ref[...], ws_ref[...],
                  preferred_element_type=jnp.float32) + pp_ref[0, 3:4, :]
    inv_n = jnp.float32(1.0 / N)
    for r in range(R):
        mu = st_ref[r, 0:1, :] * inv_n
        var = st_ref[r, 1:2, :] * inv_n - mu * mu
        scale = lax.rsqrt(var + BN_EPS) * pp_ref[r, 0:1, :]
        hn = (h_ref[:, r, :] - mu) * scale + pp_ref[r, 1:2, :]
        hn = jnp.maximum(hn, 0.0)
        acc = acc + jnp.dot(hn, w2_ref[r],
                            preferred_element_type=jnp.float32)
        acc = acc + pp_ref[r, 2:3, :]
    o_ref[...] = acc


def kernel(x, edge_index, edge_type, W_self, b_self, W1, b1, gamma, beta,
           W2, b2):
    x = x.astype(jnp.float32)
    src = edge_index[0].astype(jnp.int32)
    dst = edge_index[1].astype(jnp.int32)
    typ = edge_type.astype(jnp.int32)

    pad = EPAD - E
    gid = dst * R + typ
    src = jnp.concatenate([src, jnp.zeros((pad,), jnp.int32)])
    gid = jnp.concatenate([gid, jnp.full((pad,), N * R, jnp.int32)])
    srcT = src.reshape(NS, EP)
    gidT = gid.reshape(NS, EP)

    zeros_hbm = jnp.zeros((ZROWS, D), jnp.float32)

    agg = _make_sc_agg()(x, srcT, gidT, zeros_hbm)
    agg3 = agg.reshape(NTOT, R, D)

    b1r = b1.reshape(R, 1, D)
    h, stats = pl.pallas_call(
        _tc_stats_kernel,
        grid=(NB,),
        in_specs=[
            pl.BlockSpec((BLK, D), lambda nb: (nb, 0)),
            pl.BlockSpec((BLK, R, D), lambda nb: (nb, 0, 0)),
            pl.BlockSpec((R, D, D), lambda nb: (0, 0, 0)),
            pl.BlockSpec((R, 1, D), lambda nb: (0, 0, 0)),
        ],
        out_specs=[
            pl.BlockSpec((BLK, R, D), lambda nb: (nb, 0, 0)),
            pl.BlockSpec((R, 8, D), lambda nb: (0, 0, 0)),
        ],
        out_shape=[
            jax.ShapeDtypeStruct((N, R, D), jnp.float32),
            jax.ShapeDtypeStruct((R, 8, D), jnp.float32),
        ],
    )(x, agg3, W1, b1r)

    params = jnp.zeros((R, 8, D), jnp.float32)
    params = params.at[:, 0, :].set(gamma)
    params = params.at[:, 1, :].set(beta)
    params = params.at[:, 2, :].set(b2)
    params = params.at[0, 3, :].set(b_self)

    out = pl.pallas_call(
        _tc_final_kernel,
        grid=(NB,),
        in_specs=[
            pl.BlockSpec((BLK, D), lambda nb: (nb, 0)),
            pl.BlockSpec((BLK, R, D), lambda nb: (nb, 0, 0)),
            pl.BlockSpec((R, 8, D), lambda nb: (0, 0, 0)),
            pl.BlockSpec((D, D), lambda nb: (0, 0)),
            pl.BlockSpec((R, D, D), lambda nb: (0, 0, 0)),
            pl.BlockSpec((R, 8, D), lambda nb: (0, 0, 0)),
        ],
        out_specs=pl.BlockSpec((BLK, D), lambda nb: (nb, 0)),
        out_shape=jax.ShapeDtypeStruct((N, D), jnp.float32),
    )(x, h, stats, W_self, W2, params)

    return out


# speedup vs baseline: 1.0812x; 1.0404x over previous
import functools

import jax
import jax.numpy as jnp
from jax import lax
from jax.experimental import pallas as pl
from jax.experimental.pallas import tpu as pltpu
from jax.experimental.pallas import tpu_sc as plsc

N = 10000
D = 128
R = 4
E = 320000
BN_EPS = 1e-5

NC = 2
NS = 16
L = 16

NCHUNK = 2560
NTOT = 4 * NCHUNK
B = 64
NSLOT = 3
SB = 512
NSB = 40
EP = NSB * SB
EPAD = NS * EP
GBASE = NCHUNK * R
SH_ROWS = GBASE + 64
ZROWS = SH_ROWS // NS
SHIFT = 14


def _lane_take(v, idx):
    return lax.gather(
        v, idx[:, None],
        lax.GatherDimensionNumbers(
            offset_dims=(), collapsed_slice_dims=(0,), start_index_map=(0,)),
        slice_sizes=(1,),
        mode=lax.GatherScatterMode.PROMISE_IN_BOUNDS)


def _sc_agg_kernel(x_hbm, src_hbm, gid_hbm, z_hbm, out_hbm,
                   srcm0, srcm1, gidm0, gidm1, packedc, rows0, rows1, rows2,
                   gb0, gb1, gb2, sb0, sb1, sb2, shared,
                   semm0, semm1, semg0, semg1, semg2, sems0, sems1, sems2):
    c = lax.axis_index("c")
    s = lax.axis_index("s")
    srcm = (srcm0, srcm1)
    gidm = (gidm0, gidm1)
    rows = (rows0, rows1, rows2)
    gbuf = (gb0, gb1, gb2)
    sbuf = (sb0, sb1, sb2)
    semm = (semm0, semm1)
    semg = (semg0, semg1, semg2)
    sems = (sems0, sems1, sems2)
    iota = lax.iota(jnp.int32, L)

    def meta_start(sb, par):
        pltpu.async_copy(src_hbm.at[s, pl.ds(sb * SB, SB)], srcm[par],
                         semm[par])
        pltpu.async_copy(gid_hbm.at[s, pl.ds(sb * SB, SB)], gidm[par],
                         semm[par])

    def meta_wait(par):
        pltpu.make_async_copy(src_hbm.at[0, pl.ds(0, SB)], srcm[par],
                              semm[par]).wait()
        pltpu.make_async_copy(gid_hbm.at[0, pl.ds(0, SB)], gidm[par],
                              semm[par]).wait()

    def gather_start(jpar):
        pltpu.async_copy(x_hbm.at[gbuf[jpar].at[0]], rows[jpar], semg[jpar])

    def gather_wait(jpar):
        pltpu.make_async_copy(x_hbm.at[gbuf[jpar].at[0]], rows[jpar],
                              semg[jpar]).wait()

    def prep(b, jpar):
        for jj in range(B // L):
            pv = packedc[pl.ds(b * B + jj * L, L)]
            gbuf[jpar][0, pl.ds(jj * L, L)] = lax.shift_right_logical(
                pv, SHIFT)
            sbuf[jpar][0, pl.ds(jj * L, L)] = pv & ((1 << SHIFT) - 1)

    def scatter_start(jpar):
        pltpu.async_copy(rows[jpar], shared.at[sbuf[jpar].at[0]], sems[jpar],
                         add=True)

    def scatter_wait(jpar):
        pltpu.make_async_copy(rows[jpar], shared.at[sbuf[jpar].at[0]],
                              sems[jpar]).wait()

    def run_pass(lo4, acc_rows, out_base):
        orows = acc_rows // NS
        pltpu.sync_copy(z_hbm, shared.at[pl.ds(s * ZROWS, ZROWS)])
        plsc.subcore_barrier()

        meta_start(0, 0)

        def compact_outer(q, fill):
            for par in (0, 1):
                sb = 2 * q + par
                meta_wait(par)
                if par == 0:
                    meta_start(sb + 1, 1)
                else:
                    @pl.when(q < NSB // 2 - 1)
                    def _():
                        meta_start(sb + 1, 0)
                def one_group(g):
                    gv = gidm[par][pl.ds(g * L, L)]
                    sv = srcm[par][pl.ds(g * L, L)]
                    rel = gv - lo4
                    inr = (rel >= 0) & (rel < acc_rows)
                    packed = lax.shift_left(sv, SHIFT) | (
                        rel & ((1 << SHIFT) - 1))
                    ps = jnp.where(inr, jnp.full((L,), 1, jnp.int32),
                                   jnp.full((L,), 0, jnp.int32))
                    for kk in (1, 2, 4, 8):
                        sh = _lane_take(ps, jnp.maximum(iota - kk, 0))
                        ps = ps + jnp.where(iota >= kk, sh, 0)
                    tgt = iota + 1
                    lo = jnp.zeros((L,), jnp.int32)
                    for kk in (8, 4, 2, 1):
                        sm = _lane_take(ps, lo + (kk - 1))
                        lo = jnp.where(sm < tgt, lo + kk, lo)
                    perm = jnp.minimum(lo, L - 1)
                    return _lane_take(packed, perm), ps

                def group_body(q2, fill):
                    g = 2 * q2
                    comp0, ps0 = one_group(g)
                    comp1, ps1 = one_group(g + 1)
                    packedc[pl.ds(fill, L)] = comp0
                    fill = fill + ps0[15]
                    packedc[pl.ds(fill, L)] = comp1
                    return fill + ps1[15]

                fill = lax.fori_loop(0, SB // L // 2, group_body, fill)
            return fill

        fill = lax.fori_loop(0, NSB // 2, compact_outer, jnp.int32(0))

        for jj in range(B // L):
            packedc[pl.ds(fill + jj * L, L)] = (
                GBASE + jj * L + iota)

        nb = lax.div(fill + (B - 1), B)

        def ring(g, carry):
            b0 = NSLOT * g
            for t in range(NSLOT):
                b = b0 + t

                @pl.when(b < nb)
                def _():
                    @pl.when(b >= NSLOT)
                    def _():
                        scatter_wait(t)

                    prep(b, t)
                    gather_start(t)

            for t in range(NSLOT):
                b = b0 + t

                @pl.when(b < nb)
                def _():
                    gather_wait(t)
                    scatter_start(t)

            return carry

        lax.fori_loop(0, lax.div(nb + (NSLOT - 1), NSLOT), ring, 0)
        for t in range(NSLOT):
            @pl.when(nb > t)
            def _():
                scatter_wait(t)
        plsc.subcore_barrier()

        pltpu.sync_copy(
            shared.at[pl.ds(s * orows, orows)],
            out_hbm.at[pl.ds(out_base + s * orows, orows)])
        plsc.subcore_barrier()

    for p in range(2):
        k = 2 * c + p
        run_pass(k * (NCHUNK * R), NCHUNK * R, k * (NCHUNK * R))


@functools.lru_cache(maxsize=None)
def _make_sc_agg():
    return functools.partial(
        pl.kernel,
        out_type=jax.ShapeDtypeStruct((NTOT * R, D), jnp.float32),
        mesh=plsc.VectorSubcoreMesh(core_axis_name="c", subcore_axis_name="s"),
        scratch_types=[
            pltpu.VMEM((SB,), jnp.int32),
            pltpu.VMEM((SB,), jnp.int32),
            pltpu.VMEM((SB,), jnp.int32),
            pltpu.VMEM((SB,), jnp.int32),
            pltpu.VMEM((EP + B,), jnp.int32),
            pltpu.VMEM((B, D), jnp.float32),
            pltpu.VMEM((B, D), jnp.float32),
            pltpu.VMEM((B, D), jnp.float32),
            pltpu.VMEM((1, B), jnp.int32),
            pltpu.VMEM((1, B), jnp.int32),
            pltpu.VMEM((1, B), jnp.int32),
            pltpu.VMEM((1, B), jnp.int32),
            pltpu.VMEM((1, B), jnp.int32),
            pltpu.VMEM((1, B), jnp.int32),
            pltpu.VMEM_SHARED((SH_ROWS, D), jnp.float32),
            pltpu.SemaphoreType.DMA,
            pltpu.SemaphoreType.DMA,
            pltpu.SemaphoreType.DMA,
            pltpu.SemaphoreType.DMA,
            pltpu.SemaphoreType.DMA,
            pltpu.SemaphoreType.DMA,
            pltpu.SemaphoreType.DMA,
            pltpu.SemaphoreType.DMA,
        ],
    )(_sc_agg_kernel)


BLK = 400
NB = N // BLK


def _tc_stats_kernel(x_ref, agg_ref, w1_ref, b1_ref, h_ref, st_ref):
    nb = pl.program_id(0)
    xb = x_ref[...]
    parts = []
    for r in range(R):
        xa = xb + agg_ref[:, r, :]
        h = jnp.dot(xa, w1_ref[r], preferred_element_type=jnp.float32) \
            + b1_ref[r]
        h_ref[:, r, :] = h
        ssum = jnp.sum(h, axis=0, keepdims=True)
        ssq = jnp.sum(h * h, axis=0, keepdims=True)
        parts.append(jnp.concatenate(
            [ssum, ssq, jnp.zeros((6, D), jnp.float32)], axis=0))
    contrib = jnp.stack(parts, axis=0)

    @pl.when(nb == 0)
    def _():
        st_ref[...] = contrib

    @pl.when(nb > 0)
    def _():
        st_ref[...] = st_ref[...] + contrib


def _tc_final_kernel(x_ref, h_ref, st_ref, ws_ref, w2_ref, pp_ref, o_ref):
    acc = jnp.dot(x_ref[...], ws_ref[...],
                  preferred_element_type=jnp.float32) + pp_ref[0, 3:4, :]
    inv_n = jnp.float32(1.0 / N)
    for r in range(R):
        mu = st_ref[r, 0:1, :] * inv_n
        var = st_ref[r, 1:2, :] * inv_n - mu * mu
        scale = lax.rsqrt(var + BN_EPS) * pp_ref[r, 0:1, :]
        hn = (h_ref[:, r, :] - mu) * scale + pp_ref[r, 1:2, :]
        hn = jnp.maximum(hn, 0.0)
        acc = acc + jnp.dot(hn, w2_ref[r],
                            preferred_element_type=jnp.float32)
        acc = acc + pp_ref[r, 2:3, :]
    o_ref[...] = acc


def kernel(x, edge_index, edge_type, W_self, b_self, W1, b1, gamma, beta,
           W2, b2):
    x = x.astype(jnp.float32)
    src = edge_index[0].astype(jnp.int32)
    dst = edge_index[1].astype(jnp.int32)
    typ = edge_type.astype(jnp.int32)

    pad = EPAD - E
    gid = dst * R + typ
    src = jnp.concatenate([src, jnp.zeros((pad,), jnp.int32)])
    gid = jnp.concatenate([gid, jnp.full((pad,), N * R, jnp.int32)])
    srcT = src.reshape(NS, EP)
    gidT = gid.reshape(NS, EP)

    zeros_hbm = jnp.zeros((ZROWS, D), jnp.float32)

    agg = _make_sc_agg()(x, srcT, gidT, zeros_hbm)
    agg3 = agg.reshape(NTOT, R, D)

    b1r = b1.reshape(R, 1, D)
    h, stats = pl.pallas_call(
        _tc_stats_kernel,
        grid=(NB,),
        in_specs=[
            pl.BlockSpec((BLK, D), lambda nb: (nb, 0)),
            pl.BlockSpec((BLK, R, D), lambda nb: (nb, 0, 0)),
            pl.BlockSpec((R, D, D), lambda nb: (0, 0, 0)),
            pl.BlockSpec((R, 1, D), lambda nb: (0, 0, 0)),
        ],
        out_specs=[
            pl.BlockSpec((BLK, R, D), lambda nb: (nb, 0, 0)),
            pl.BlockSpec((R, 8, D), lambda nb: (0, 0, 0)),
        ],
        out_shape=[
            jax.ShapeDtypeStruct((N, R, D), jnp.float32),
            jax.ShapeDtypeStruct((R, 8, D), jnp.float32),
        ],
    )(x, agg3, W1, b1r)

    params = jnp.zeros((R, 8, D), jnp.float32)
    params = params.at[:, 0, :].set(gamma)
    params = params.at[:, 1, :].set(beta)
    params = params.at[:, 2, :].set(b2)
    params = params.at[0, 3, :].set(b_self)

    out = pl.pallas_call(
        _tc_final_kernel,
        grid=(NB,),
        in_specs=[
            pl.BlockSpec((BLK, D), lambda nb: (nb, 0)),
            pl.BlockSpec((BLK, R, D), lambda nb: (nb, 0, 0)),
            pl.BlockSpec((R, 8, D), lambda nb: (0, 0, 0)),
            pl.BlockSpec((D, D), lambda nb: (0, 0)),
            pl.BlockSpec((R, D, D), lambda nb: (0, 0, 0)),
            pl.BlockSpec((R, 8, D), lambda nb: (0, 0, 0)),
        ],
        out_specs=pl.BlockSpec((BLK, D), lambda nb: (nb, 0)),
        out_shape=jax.ShapeDtypeStruct((N, D), jnp.float32),
    )(x, h, stats, W_self, W2, params)

    return out
